# Initial kernel scaffold; baseline (speedup 1.0000x reference)
import functools

import jax
import jax.numpy as jnp
from jax import lax
from jax.experimental import pallas as pl
from jax.experimental.pallas import tpu as pltpu
from jax.experimental.pallas import tpu_sc as plsc

N = 10000
E = 320000
D = 128
NC = 2
NS = 16
NW = NC * NS
EPW = E // NW
C = 80
NCH = EPW // C
RPT = 313
NPAD = RPT * NS
NPAD = RPT * NW // NC * NC
BR = 1250
GRID = N // BR


def _leaky(v):
    return jnp.where(v > 0, v, 0.1 * v)


_sc_mesh = plsc.VectorSubcoreMesh(core_axis_name="c", subcore_axis_name="s")


@functools.partial(
    pl.kernel,
    out_type=(
        jax.ShapeDtypeStruct((NPAD, D), jnp.float32),
        jax.ShapeDtypeStruct((NPAD, D), jnp.float32),
    ),
    mesh=_sc_mesh,
    scratch_types=dict(
        src_v=pltpu.VMEM((C,), jnp.int32),
        dst_v=pltpu.VMEM((C,), jnp.int32),
        rows_v=pltpu.VMEM((C, D), jnp.float32),
        acc_sh=pltpu.VMEM_SHARED((NPAD, D), jnp.float32),
        gsem=pltpu.SemaphoreType.DMA,
    ),
)
def _sc_segsum(g_hbm, src_hbm, dst_hbm, zeros_hbm, out0, out1,
               src_v, dst_v, rows_v, acc_sh, gsem):
    cid = lax.axis_index("c")
    sid = lax.axis_index("s")
    wid = sid * NC + cid

    r0 = sid * 2 * RPT
    pltpu.sync_copy(zeros_hbm.at[pl.ds(r0, 2 * RPT)], acc_sh.at[pl.ds(r0, 2 * RPT)])
    plsc.subcore_barrier()

    def body(i, carry):
        base = wid * EPW + i * C
        pltpu.sync_copy(src_hbm.at[pl.ds(base, C)], src_v)
        pltpu.sync_copy(dst_hbm.at[pl.ds(base, C)], dst_v)
        pltpu.async_copy(g_hbm.at[src_v], rows_v, gsem).wait()
        pltpu.sync_copy(rows_v, acc_sh.at[dst_v], add=True)
        return carry

    lax.fori_loop(0, NCH, body, 0)
    plsc.subcore_barrier()

    @pl.when(cid == 0)
    def _():
        pltpu.sync_copy(acc_sh.at[pl.ds(r0, 2 * RPT)], out0.at[pl.ds(r0, 2 * RPT)])

    @pl.when(cid == 1)
    def _():
        pltpu.sync_copy(acc_sh.at[pl.ds(r0, 2 * RPT)], out1.at[pl.ds(r0, 2 * RPT)])


def _mm_body(x_ref, w_ref, o_ref):
    o_ref[...] = jnp.dot(x_ref[...], w_ref[...], preferred_element_type=jnp.float32)


def _tc_matmul(x, w):
    return pl.pallas_call(
        _mm_body,
        grid=(GRID,),
        in_specs=[
            pl.BlockSpec((BR, D), lambda i: (i, 0)),
            pl.BlockSpec((D, D), lambda i: (0, 0)),
        ],
        out_specs=pl.BlockSpec((BR, D), lambda i: (i, 0)),
        out_shape=jax.ShapeDtypeStruct((N, D), jnp.float32),
    )(x, w)


def _fuse_body(p0_ref, p1_ref, b_ref, w_ref, h_ref, g_ref):
    h = _leaky(p0_ref[...] + p1_ref[...] + b_ref[...])
    h_ref[...] = h
    g_ref[...] = jnp.dot(h, w_ref[...], preferred_element_type=jnp.float32)


def _tc_fuse(p0, p1, b, w):
    return pl.pallas_call(
        _fuse_body,
        grid=(GRID,),
        in_specs=[
            pl.BlockSpec((BR, D), lambda i: (i, 0)),
            pl.BlockSpec((BR, D), lambda i: (i, 0)),
            pl.BlockSpec((1, D), lambda i: (0, 0)),
            pl.BlockSpec((D, D), lambda i: (0, 0)),
        ],
        out_specs=[
            pl.BlockSpec((BR, D), lambda i: (i, 0)),
            pl.BlockSpec((BR, D), lambda i: (i, 0)),
        ],
        out_shape=[
            jax.ShapeDtypeStruct((N, D), jnp.float32),
            jax.ShapeDtypeStruct((N, D), jnp.float32),
        ],
    )(p0, p1, b.reshape(1, D), w)


def _final_body(p0_ref, p1_ref, b2_ref, x_ref, h1_ref, h2_ref, wo_ref, bo_ref,
                o_ref):
    h3 = _leaky(p0_ref[...] + p1_ref[...] + b2_ref[...])
    wo = wo_ref[...]
    acc = jnp.dot(x_ref[...], wo[0:D], preferred_element_type=jnp.float32)
    acc += jnp.dot(h1_ref[...], wo[D:2 * D], preferred_element_type=jnp.float32)
    acc += jnp.dot(h2_ref[...], wo[2 * D:3 * D], preferred_element_type=jnp.float32)
    acc += jnp.dot(h3, wo[3 * D:4 * D], preferred_element_type=jnp.float32)
    o_ref[...] = _leaky(acc + bo_ref[...])


def _tc_final(p0, p1, b2, x, h1, h2, wout, bout):
    row = pl.BlockSpec((BR, D), lambda i: (i, 0))
    return pl.pallas_call(
        _final_body,
        grid=(GRID,),
        in_specs=[
            row, row,
            pl.BlockSpec((1, D), lambda i: (0, 0)),
            row, row, row,
            pl.BlockSpec((4 * D, D), lambda i: (0, 0)),
            pl.BlockSpec((1, D), lambda i: (0, 0)),
        ],
        out_specs=row,
        out_shape=jax.ShapeDtypeStruct((N, D), jnp.float32),
    )(p0, p1, b2.reshape(1, D), x, h1, h2, wout, bout.reshape(1, D))


def kernel(x, edge_index, W0, b0, W1, b1, W2, b2, Wout, bout):
    src = edge_index[0]
    dst = edge_index[1]
    zeros = jnp.zeros((NPAD, D), jnp.float32)

    g0 = _tc_matmul(x, W0)
    p0a, p0b = _sc_segsum(g0, src, dst, zeros)
    h1, g1 = _tc_fuse(p0a, p0b, b0, W1)
    p1a, p1b = _sc_segsum(g1, src, dst, zeros)
    h2, g2 = _tc_fuse(p1a, p1b, b1, W2)
    p2a, p2b = _sc_segsum(g2, src, dst, zeros)
    return _tc_final(p2a, p2b, b2, x, h1, h2, Wout, bout)


# speedup vs baseline: 4.9183x; 4.9183x over previous
import functools

import jax
import jax.numpy as jnp
from jax import lax
from jax.experimental import pallas as pl
from jax.experimental.pallas import tpu as pltpu
from jax.experimental.pallas import tpu_sc as plsc

N = 10000
E = 320000
D = 128
NC = 2
NS = 16
NW = NC * NS
EPW = E // NW
C = 80
NCH = EPW // C
RPT = 632
NPAD = RPT * NS
BR = 1000
GRID = N // BR


def _leaky(v):
    return jnp.where(v > 0, v, 0.1 * v)


@functools.cache
def _get_sc_segsum():
    mesh = plsc.VectorSubcoreMesh(core_axis_name="c", subcore_axis_name="s")
    return functools.partial(
        pl.kernel,
        out_type=(
            jax.ShapeDtypeStruct((NPAD, D), jnp.float32),
            jax.ShapeDtypeStruct((NPAD, D), jnp.float32),
        ),
        mesh=mesh,
        scratch_types=dict(
            src_v=pltpu.VMEM((C,), jnp.int32),
            dst_v=pltpu.VMEM((C,), jnp.int32),
            rows_v=pltpu.VMEM((C, D), jnp.float32),
            acc_sh=pltpu.VMEM_SHARED((NPAD, D), jnp.float32),
            gsem=pltpu.SemaphoreType.DMA,
        ),
    )(_sc_segsum_body)


def _sc_segsum(g, src, dst, zeros):
    return _get_sc_segsum()(g, src, dst, zeros)


def _sc_segsum_body(g_hbm, src_hbm, dst_hbm, zeros_hbm, out0, out1,
                    src_v, dst_v, rows_v, acc_sh, gsem):
    cid = lax.axis_index("c")
    sid = lax.axis_index("s")
    wid = sid * NC + cid

    r0 = sid * RPT
    pltpu.sync_copy(zeros_hbm.at[pl.ds(r0, RPT)], acc_sh.at[pl.ds(r0, RPT)])
    plsc.subcore_barrier()

    def body(i, carry):
        base = wid * EPW + i * C
        pltpu.sync_copy(src_hbm.at[pl.ds(base, C)], src_v)
        pltpu.sync_copy(dst_hbm.at[pl.ds(base, C)], dst_v)
        pltpu.async_copy(g_hbm.at[src_v], rows_v, gsem).wait()
        pltpu.sync_copy(rows_v, acc_sh.at[dst_v], add=True)
        return carry

    lax.fori_loop(0, NCH, body, 0)
    plsc.subcore_barrier()

    @pl.when(cid == 0)
    def _():
        pltpu.sync_copy(acc_sh.at[pl.ds(r0, RPT)], out0.at[pl.ds(r0, RPT)])

    @pl.when(cid == 1)
    def _():
        pltpu.sync_copy(acc_sh.at[pl.ds(r0, RPT)], out1.at[pl.ds(r0, RPT)])


def _mm_body(x_ref, w_ref, o_ref):
    o_ref[...] = jnp.dot(x_ref[...], w_ref[...], preferred_element_type=jnp.float32)


def _tc_matmul(x, w):
    return pl.pallas_call(
        _mm_body,
        grid=(GRID,),
        in_specs=[
            pl.BlockSpec((BR, D), lambda i: (i, 0)),
            pl.BlockSpec((D, D), lambda i: (0, 0)),
        ],
        out_specs=pl.BlockSpec((BR, D), lambda i: (i, 0)),
        out_shape=jax.ShapeDtypeStruct((N, D), jnp.float32),
    )(x, w)


def _fuse_body(p0_ref, p1_ref, b_ref, w_ref, h_ref, g_ref):
    h = _leaky(p0_ref[...] + p1_ref[...] + b_ref[...])
    h_ref[...] = h
    g_ref[...] = jnp.dot(h, w_ref[...], preferred_element_type=jnp.float32)


def _tc_fuse(p0, p1, b, w):
    return pl.pallas_call(
        _fuse_body,
        grid=(GRID,),
        in_specs=[
            pl.BlockSpec((BR, D), lambda i: (i, 0)),
            pl.BlockSpec((BR, D), lambda i: (i, 0)),
            pl.BlockSpec((1, D), lambda i: (0, 0)),
            pl.BlockSpec((D, D), lambda i: (0, 0)),
        ],
        out_specs=[
            pl.BlockSpec((BR, D), lambda i: (i, 0)),
            pl.BlockSpec((BR, D), lambda i: (i, 0)),
        ],
        out_shape=[
            jax.ShapeDtypeStruct((N, D), jnp.float32),
            jax.ShapeDtypeStruct((N, D), jnp.float32),
        ],
    )(p0, p1, b.reshape(1, D), w)


def _final_body(p0_ref, p1_ref, b2_ref, x_ref, h1_ref, h2_ref, wo_ref, bo_ref,
                o_ref):
    h3 = _leaky(p0_ref[...] + p1_ref[...] + b2_ref[...])
    wo = wo_ref[...]
    acc = jnp.dot(x_ref[...], wo[0:D], preferred_element_type=jnp.float32)
    acc += jnp.dot(h1_ref[...], wo[D:2 * D], preferred_element_type=jnp.float32)
    acc += jnp.dot(h2_ref[...], wo[2 * D:3 * D], preferred_element_type=jnp.float32)
    acc += jnp.dot(h3, wo[3 * D:4 * D], preferred_element_type=jnp.float32)
    o_ref[...] = _leaky(acc + bo_ref[...])


def _tc_final(p0, p1, b2, x, h1, h2, wout, bout):
    row = pl.BlockSpec((BR, D), lambda i: (i, 0))
    return pl.pallas_call(
        _final_body,
        grid=(GRID,),
        in_specs=[
            row, row,
            pl.BlockSpec((1, D), lambda i: (0, 0)),
            row, row, row,
            pl.BlockSpec((4 * D, D), lambda i: (0, 0)),
            pl.BlockSpec((1, D), lambda i: (0, 0)),
        ],
        out_specs=row,
        out_shape=jax.ShapeDtypeStruct((N, D), jnp.float32),
    )(p0, p1, b2.reshape(1, D), x, h1, h2, wout, bout.reshape(1, D))


def kernel(x, edge_index, W0, b0, W1, b1, W2, b2, Wout, bout):
    src = edge_index[0]
    dst = edge_index[1]
    zeros = jnp.zeros((NPAD, D), jnp.float32)

    g0 = _tc_matmul(x, W0)
    p0a, p0b = _sc_segsum(g0, src, dst, zeros)
    h1, g1 = _tc_fuse(p0a, p0b, b0, W1)
    p1a, p1b = _sc_segsum(g1, src, dst, zeros)
    h2, g2 = _tc_fuse(p1a, p1b, b1, W2)
    p2a, p2b = _sc_segsum(g2, src, dst, zeros)
    return _tc_final(p2a, p2b, b2, x, h1, h2, Wout, bout)
